# baseline (speedup 1.0000x reference)
import functools

import jax
import jax.numpy as jnp
from jax import lax
from jax.experimental import pallas as pl
from jax.experimental.pallas import tpu as pltpu
from jax.experimental.pallas import tpu_sc as plsc

_D = 32
_K = 8192
_BM = 128


def _dist_kernel(x_ref, cb_ref, a2_ref, b2_ref, idx_ref, loss_ref, acc_ref,
                 *, num_m):
    m = pl.program_id(0)
    x = x_ref[...]
    cb = cb_ref[...]
    p = lax.dot_general(x, cb, (((1,), (1,)), ((), ())),
                        preferred_element_type=jnp.float32,
                        precision=lax.Precision.HIGHEST)
    d2 = (a2_ref[...] + b2_ref[...]) - 2.0 * p
    d2 = jnp.maximum(d2, 0.0)
    d = jnp.where(d2 == 0.0, 0.0, d2 * lax.rsqrt(d2))
    bmin = jnp.min(d, axis=1, keepdims=True)
    iota = lax.broadcasted_iota(jnp.int32, d.shape, 1)
    bidx = jnp.min(jnp.where(d == bmin, iota, jnp.int32(2**30)),
                   axis=1, keepdims=True)
    idx_ref[...] = bidx
    part = jnp.sum(bmin * bmin)

    @pl.when(m == 0)
    def _():
        acc_ref[0] = part

    @pl.when(m > 0)
    def _():
        acc_ref[0] += part

    @pl.when(m == num_m - 1)
    def _():
        loss_ref[0, 0] = acc_ref[0]


def _dist_argmin(flat, codebook, a2, b2):
    m_tokens = flat.shape[0]
    num_m = m_tokens // _BM
    return pl.pallas_call(
        functools.partial(_dist_kernel, num_m=num_m),
        grid=(num_m,),
        in_specs=[
            pl.BlockSpec((_BM, _D), lambda m: (m, 0)),
            pl.BlockSpec((_K, _D), lambda m: (0, 0)),
            pl.BlockSpec((_BM, 1), lambda m: (m, 0)),
            pl.BlockSpec((1, _K), lambda m: (0, 0)),
        ],
        out_specs=[
            pl.BlockSpec((_BM, 1), lambda m: (m, 0)),
            pl.BlockSpec((1, 1), lambda m: (0, 0),
                         memory_space=pltpu.SMEM),
        ],
        out_shape=[
            jax.ShapeDtypeStruct((m_tokens, 1), jnp.int32),
            jax.ShapeDtypeStruct((1, 1), jnp.float32),
        ],
        scratch_shapes=[pltpu.SMEM((1,), jnp.float32)],
    )(flat, codebook, a2, b2)


def _sc_gather(table, idx):
    info = plsc.get_sparse_core_info()
    nc, ns = info.num_cores, info.num_subcores
    nw = nc * ns
    b = idx.shape[0]
    d = table.shape[1]
    bw = b // nw
    nchunk = bw // 128
    idx2d = idx.reshape(b // 128, 128)
    mesh = plsc.VectorSubcoreMesh(core_axis_name="c", subcore_axis_name="s")

    @functools.partial(
        pl.kernel,
        mesh=mesh,
        compiler_params=pltpu.CompilerParams(use_tc_tiling_on_sc=False),
        out_type=jax.ShapeDtypeStruct((b, d), table.dtype),
        scratch_types=[
            pltpu.VMEM((nchunk, 128), jnp.int32),
            pltpu.VMEM((bw, d), jnp.float32),
            pltpu.SemaphoreType.DMA,
        ],
    )
    def gk(table_hbm, idx_hbm, out_hbm, idx_v, rows_v, sem):
        wid = lax.axis_index("s") * nc + lax.axis_index("c")
        pltpu.sync_copy(idx_hbm.at[pl.ds(wid * nchunk, nchunk)], idx_v)
        copies = [
            pltpu.async_copy(table_hbm.at[idx_v.at[j]],
                             rows_v.at[pl.ds(j * 128, 128)], sem)
            for j in range(nchunk)
        ]
        for c in copies:
            c.wait()
        pltpu.sync_copy(rows_v, out_hbm.at[pl.ds(wid * bw, bw)])

    return gk(table, idx2d)


def kernel(inputs, codebook):
    x = jnp.transpose(inputs, (0, 2, 3, 1))
    flat = x.reshape(-1, _D)
    a2 = jnp.sum(flat * flat, axis=1, keepdims=True)
    b2 = jnp.sum(codebook * codebook, axis=1)[None, :]
    flat_bf = flat.astype(jnp.bfloat16).astype(jnp.float32)
    idx2, loss11 = _dist_argmin(flat_bf, codebook, a2, b2)
    quantized = _sc_gather(codebook, idx2.reshape(-1)).reshape(x.shape)
    out = jnp.transpose(quantized, (0, 3, 1, 2))
    mean = loss11[0, 0] / jnp.float32(flat.shape[0] * _D)
    commitment_loss = mean + 0.25 * mean
    return out, commitment_loss


# speedup vs baseline: 2.4801x; 2.4801x over previous
import functools

import jax
import jax.numpy as jnp
from jax import lax
from jax.experimental import pallas as pl
from jax.experimental.pallas import tpu as pltpu
from jax.experimental.pallas import tpu_sc as plsc

_D = 32
_K = 8192
_BM = 256


def _dist_kernel(x_ref, cb_ref, a2_ref, b2_ref, idx_ref, loss_ref, acc_ref,
                 *, num_m):
    m = pl.program_id(0)
    x = x_ref[...]
    cb = cb_ref[...]
    p = lax.dot_general(x, cb, (((1,), (1,)), ((), ())),
                        preferred_element_type=jnp.float32)
    d2 = (a2_ref[...] + b2_ref[...]) - 2.0 * p
    d2 = jnp.maximum(d2, 0.0)
    bmin = jnp.min(d2, axis=1, keepdims=True)
    iota = lax.broadcasted_iota(jnp.int32, d2.shape, 1)
    bidx = jnp.min(jnp.where(d2 == bmin, iota, jnp.int32(2**30)),
                   axis=1, keepdims=True)
    idx_ref[...] = bidx
    part = jnp.sum(bmin)

    @pl.when(m == 0)
    def _():
        acc_ref[0] = part

    @pl.when(m > 0)
    def _():
        acc_ref[0] += part

    @pl.when(m == num_m - 1)
    def _():
        loss_ref[0, 0] = acc_ref[0]


def _dist_argmin(flat, codebook, a2, b2):
    m_tokens = flat.shape[0]
    num_m = m_tokens // _BM
    return pl.pallas_call(
        functools.partial(_dist_kernel, num_m=num_m),
        grid=(num_m,),
        in_specs=[
            pl.BlockSpec((_BM, _D), lambda m: (m, 0)),
            pl.BlockSpec((_K, _D), lambda m: (0, 0)),
            pl.BlockSpec((_BM, 1), lambda m: (m, 0)),
            pl.BlockSpec((1, _K), lambda m: (0, 0)),
        ],
        out_specs=[
            pl.BlockSpec((_BM, 1), lambda m: (m, 0)),
            pl.BlockSpec((1, 1), lambda m: (0, 0),
                         memory_space=pltpu.SMEM),
        ],
        out_shape=[
            jax.ShapeDtypeStruct((m_tokens, 1), jnp.int32),
            jax.ShapeDtypeStruct((1, 1), jnp.float32),
        ],
        scratch_shapes=[pltpu.SMEM((1,), jnp.float32)],
    )(flat, codebook, a2, b2)


def _sc_gather(table, idx):
    info = plsc.get_sparse_core_info()
    nc, ns = info.num_cores, info.num_subcores
    nw = nc * ns
    b = idx.shape[0]
    d = table.shape[1]
    bw = b // nw
    nchunk = bw // 128
    idx2d = idx.reshape(b // 128, 128)
    mesh = plsc.VectorSubcoreMesh(core_axis_name="c", subcore_axis_name="s")

    @functools.partial(
        pl.kernel,
        mesh=mesh,
        compiler_params=pltpu.CompilerParams(use_tc_tiling_on_sc=False),
        out_type=jax.ShapeDtypeStruct((b, d), table.dtype),
        scratch_types=[
            pltpu.VMEM((nchunk, 128), jnp.int32),
            pltpu.VMEM((bw, d), jnp.float32),
            pltpu.SemaphoreType.DMA,
        ],
    )
    def gk(table_hbm, idx_hbm, out_hbm, idx_v, rows_v, sem):
        wid = lax.axis_index("s") * nc + lax.axis_index("c")
        pltpu.sync_copy(idx_hbm.at[pl.ds(wid * nchunk, nchunk)], idx_v)
        copies = [
            pltpu.async_copy(table_hbm.at[idx_v.at[j]],
                             rows_v.at[pl.ds(j * 128, 128)], sem)
            for j in range(nchunk)
        ]
        for c in copies:
            c.wait()
        pltpu.sync_copy(rows_v, out_hbm.at[pl.ds(wid * bw, bw)])

    return gk(table, idx2d)


def kernel(inputs, codebook):
    x = jnp.transpose(inputs, (0, 2, 3, 1))
    flat = x.reshape(-1, _D)
    a2 = jnp.sum(flat * flat, axis=1, keepdims=True)
    b2 = jnp.sum(codebook * codebook, axis=1)[None, :]
    idx2, loss11 = _dist_argmin(flat, codebook, a2, b2)
    quantized = _sc_gather(codebook, idx2.reshape(-1)).reshape(x.shape)
    out = jnp.transpose(quantized, (0, 3, 1, 2))
    mean = loss11[0, 0] / jnp.float32(flat.shape[0] * _D)
    commitment_loss = mean + 0.25 * mean
    return out, commitment_loss


# speedup vs baseline: 2.6402x; 1.0645x over previous
import functools

import jax
import jax.numpy as jnp
from jax import lax
from jax.experimental import pallas as pl
from jax.experimental.pallas import tpu as pltpu
from jax.experimental.pallas import tpu_sc as plsc

_D = 32
_K = 8192
_BM = 512


def _dist_kernel(x_ref, cb_ref, a2_ref, b2_ref, idx_ref, loss_ref, acc_ref,
                 *, num_m):
    m = pl.program_id(0)
    x = x_ref[...]
    cb = cb_ref[...]
    p = lax.dot_general(x, cb, (((1,), (1,)), ((), ())),
                        preferred_element_type=jnp.float32)
    d2 = (a2_ref[...] + b2_ref[...]) - 2.0 * p
    d2 = jnp.maximum(d2, 0.0)
    bmin = jnp.min(d2, axis=1, keepdims=True)
    iota = lax.broadcasted_iota(jnp.int32, d2.shape, 1)
    bidx = jnp.min(jnp.where(d2 == bmin, iota, jnp.int32(2**30)),
                   axis=1, keepdims=True)
    idx_ref[...] = bidx
    part = jnp.sum(bmin)

    @pl.when(m == 0)
    def _():
        acc_ref[0] = part

    @pl.when(m > 0)
    def _():
        acc_ref[0] += part

    @pl.when(m == num_m - 1)
    def _():
        loss_ref[0, 0] = acc_ref[0]


def _dist_argmin(flat, codebook, a2, b2):
    m_tokens = flat.shape[0]
    num_m = m_tokens // _BM
    return pl.pallas_call(
        functools.partial(_dist_kernel, num_m=num_m),
        grid=(num_m,),
        in_specs=[
            pl.BlockSpec((_BM, _D), lambda m: (m, 0)),
            pl.BlockSpec((_K, _D), lambda m: (0, 0)),
            pl.BlockSpec((_BM, 1), lambda m: (m, 0)),
            pl.BlockSpec((1, _K), lambda m: (0, 0)),
        ],
        out_specs=[
            pl.BlockSpec((_BM, 1), lambda m: (m, 0)),
            pl.BlockSpec((1, 1), lambda m: (0, 0),
                         memory_space=pltpu.SMEM),
        ],
        out_shape=[
            jax.ShapeDtypeStruct((m_tokens, 1), jnp.int32),
            jax.ShapeDtypeStruct((1, 1), jnp.float32),
        ],
        scratch_shapes=[pltpu.SMEM((1,), jnp.float32)],
    )(flat, codebook, a2, b2)


def _sc_gather(table, idx):
    info = plsc.get_sparse_core_info()
    nc, ns = info.num_cores, info.num_subcores
    nw = nc * ns
    b = idx.shape[0]
    d = table.shape[1]
    bw = b // nw
    nchunk = bw // 128
    idx2d = idx.reshape(b // 128, 128)
    mesh = plsc.VectorSubcoreMesh(core_axis_name="c", subcore_axis_name="s")

    @functools.partial(
        pl.kernel,
        mesh=mesh,
        compiler_params=pltpu.CompilerParams(use_tc_tiling_on_sc=False),
        out_type=jax.ShapeDtypeStruct((b, d), table.dtype),
        scratch_types=[
            pltpu.VMEM((nchunk, 128), jnp.int32),
            pltpu.VMEM((bw, d), jnp.float32),
            pltpu.SemaphoreType.DMA,
        ],
    )
    def gk(table_hbm, idx_hbm, out_hbm, idx_v, rows_v, sem):
        wid = lax.axis_index("s") * nc + lax.axis_index("c")
        pltpu.sync_copy(idx_hbm.at[pl.ds(wid * nchunk, nchunk)], idx_v)
        copies = [
            pltpu.async_copy(table_hbm.at[idx_v.at[j]],
                             rows_v.at[pl.ds(j * 128, 128)], sem)
            for j in range(nchunk)
        ]
        for c in copies:
            c.wait()
        pltpu.sync_copy(rows_v, out_hbm.at[pl.ds(wid * bw, bw)])

    return gk(table, idx2d)


def kernel(inputs, codebook):
    x = jnp.transpose(inputs, (0, 2, 3, 1))
    flat = x.reshape(-1, _D)
    a2 = jnp.sum(flat * flat, axis=1, keepdims=True)
    b2 = jnp.sum(codebook * codebook, axis=1)[None, :]
    idx2, loss11 = _dist_argmin(flat, codebook, a2, b2)
    quantized = _sc_gather(codebook, idx2.reshape(-1)).reshape(x.shape)
    out = jnp.transpose(quantized, (0, 3, 1, 2))
    mean = loss11[0, 0] / jnp.float32(flat.shape[0] * _D)
    commitment_loss = mean + 0.25 * mean
    return out, commitment_loss


# speedup vs baseline: 2.7367x; 1.0365x over previous
import functools

import jax
import jax.numpy as jnp
from jax import lax
from jax.experimental import pallas as pl
from jax.experimental.pallas import tpu as pltpu
from jax.experimental.pallas import tpu_sc as plsc

_D = 32
_K = 8192
_BM = 1024


def _dist_kernel(x_ref, cb_ref, a2_ref, b2_ref, idx_ref, loss_ref, acc_ref,
                 *, num_m):
    m = pl.program_id(0)
    x = x_ref[...]
    cb = cb_ref[...]
    p = lax.dot_general(x, cb, (((1,), (1,)), ((), ())),
                        preferred_element_type=jnp.float32)
    d2 = (a2_ref[...] + b2_ref[...]) - 2.0 * p
    d2 = jnp.maximum(d2, 0.0)
    bmin = jnp.min(d2, axis=1, keepdims=True)
    iota = lax.broadcasted_iota(jnp.int32, d2.shape, 1)
    bidx = jnp.min(jnp.where(d2 == bmin, iota, jnp.int32(2**30)),
                   axis=1, keepdims=True)
    idx_ref[...] = bidx
    part = jnp.sum(bmin)

    @pl.when(m == 0)
    def _():
        acc_ref[0] = part

    @pl.when(m > 0)
    def _():
        acc_ref[0] += part

    @pl.when(m == num_m - 1)
    def _():
        loss_ref[0, 0] = acc_ref[0]


def _dist_argmin(flat, codebook, a2, b2):
    m_tokens = flat.shape[0]
    num_m = m_tokens // _BM
    return pl.pallas_call(
        functools.partial(_dist_kernel, num_m=num_m),
        grid=(num_m,),
        in_specs=[
            pl.BlockSpec((_BM, _D), lambda m: (m, 0)),
            pl.BlockSpec((_K, _D), lambda m: (0, 0)),
            pl.BlockSpec((_BM, 1), lambda m: (m, 0)),
            pl.BlockSpec((1, _K), lambda m: (0, 0)),
        ],
        out_specs=[
            pl.BlockSpec((_BM, 1), lambda m: (m, 0)),
            pl.BlockSpec((1, 1), lambda m: (0, 0),
                         memory_space=pltpu.SMEM),
        ],
        out_shape=[
            jax.ShapeDtypeStruct((m_tokens, 1), jnp.int32),
            jax.ShapeDtypeStruct((1, 1), jnp.float32),
        ],
        scratch_shapes=[pltpu.SMEM((1,), jnp.float32)],
    )(flat, codebook, a2, b2)


def _sc_gather(table, idx):
    info = plsc.get_sparse_core_info()
    nc, ns = info.num_cores, info.num_subcores
    nw = nc * ns
    b = idx.shape[0]
    d = table.shape[1]
    bw = b // nw
    nchunk = bw // 128
    idx2d = idx.reshape(b // 128, 128)
    mesh = plsc.VectorSubcoreMesh(core_axis_name="c", subcore_axis_name="s")

    @functools.partial(
        pl.kernel,
        mesh=mesh,
        compiler_params=pltpu.CompilerParams(use_tc_tiling_on_sc=False),
        out_type=jax.ShapeDtypeStruct((b, d), table.dtype),
        scratch_types=[
            pltpu.VMEM((nchunk, 128), jnp.int32),
            pltpu.VMEM((bw, d), jnp.float32),
            pltpu.SemaphoreType.DMA,
        ],
    )
    def gk(table_hbm, idx_hbm, out_hbm, idx_v, rows_v, sem):
        wid = lax.axis_index("s") * nc + lax.axis_index("c")
        pltpu.sync_copy(idx_hbm.at[pl.ds(wid * nchunk, nchunk)], idx_v)
        copies = [
            pltpu.async_copy(table_hbm.at[idx_v.at[j]],
                             rows_v.at[pl.ds(j * 128, 128)], sem)
            for j in range(nchunk)
        ]
        for c in copies:
            c.wait()
        pltpu.sync_copy(rows_v, out_hbm.at[pl.ds(wid * bw, bw)])

    return gk(table, idx2d)


def kernel(inputs, codebook):
    x = jnp.transpose(inputs, (0, 2, 3, 1))
    flat = x.reshape(-1, _D)
    a2 = jnp.sum(flat * flat, axis=1, keepdims=True)
    b2 = jnp.sum(codebook * codebook, axis=1)[None, :]
    idx2, loss11 = _dist_argmin(flat, codebook, a2, b2)
    quantized = _sc_gather(codebook, idx2.reshape(-1)).reshape(x.shape)
    out = jnp.transpose(quantized, (0, 3, 1, 2))
    mean = loss11[0, 0] / jnp.float32(flat.shape[0] * _D)
    commitment_loss = mean + 0.25 * mean
    return out, commitment_loss


# speedup vs baseline: 2.8937x; 1.0574x over previous
import functools

import jax
import jax.numpy as jnp
from jax import lax
from jax.experimental import pallas as pl
from jax.experimental.pallas import tpu as pltpu
from jax.experimental.pallas import tpu_sc as plsc

_D = 32
_K = 8192
_BM = 1024


def _dist_kernel(x_ref, cb_ref, a2_ref, b2_ref, idx_ref, loss_ref, acc_ref,
                 *, num_m):
    m = pl.program_id(0)
    x = x_ref[...]
    cb = cb_ref[...]
    p = lax.dot_general(x, cb, (((1,), (1,)), ((), ())),
                        preferred_element_type=jnp.float32)
    s = b2_ref[...] - 2.0 * p
    smin = jnp.min(s, axis=1, keepdims=True)
    iota = lax.broadcasted_iota(jnp.int32, s.shape, 1)
    bidx = jnp.min(jnp.where(s == smin, iota, jnp.int32(2**30)),
                   axis=1, keepdims=True)
    idx_ref[...] = bidx
    part = jnp.sum(jnp.maximum(a2_ref[...] + smin, 0.0))

    @pl.when(m == 0)
    def _():
        acc_ref[0] = part

    @pl.when(m > 0)
    def _():
        acc_ref[0] += part

    @pl.when(m == num_m - 1)
    def _():
        loss_ref[0, 0] = acc_ref[0]


def _dist_argmin(flat, codebook, a2, b2):
    m_tokens = flat.shape[0]
    num_m = m_tokens // _BM
    return pl.pallas_call(
        functools.partial(_dist_kernel, num_m=num_m),
        grid=(num_m,),
        in_specs=[
            pl.BlockSpec((_BM, _D), lambda m: (m, 0)),
            pl.BlockSpec((_K, _D), lambda m: (0, 0)),
            pl.BlockSpec((_BM, 1), lambda m: (m, 0)),
            pl.BlockSpec((1, _K), lambda m: (0, 0)),
        ],
        out_specs=[
            pl.BlockSpec((_BM, 1), lambda m: (m, 0)),
            pl.BlockSpec((1, 1), lambda m: (0, 0),
                         memory_space=pltpu.SMEM),
        ],
        out_shape=[
            jax.ShapeDtypeStruct((m_tokens, 1), jnp.int32),
            jax.ShapeDtypeStruct((1, 1), jnp.float32),
        ],
        scratch_shapes=[pltpu.SMEM((1,), jnp.float32)],
    )(flat, codebook, a2, b2)


def _sc_gather(table, idx):
    info = plsc.get_sparse_core_info()
    nc, ns = info.num_cores, info.num_subcores
    nw = nc * ns
    b = idx.shape[0]
    d = table.shape[1]
    bw = b // nw
    nchunk = bw // 128
    idx2d = idx.reshape(b // 128, 128)
    mesh = plsc.VectorSubcoreMesh(core_axis_name="c", subcore_axis_name="s")

    @functools.partial(
        pl.kernel,
        mesh=mesh,
        compiler_params=pltpu.CompilerParams(use_tc_tiling_on_sc=False),
        out_type=jax.ShapeDtypeStruct((b, d), table.dtype),
        scratch_types=[
            pltpu.VMEM((nchunk, 128), jnp.int32),
            pltpu.VMEM((bw, d), jnp.float32),
            pltpu.SemaphoreType.DMA,
        ],
    )
    def gk(table_hbm, idx_hbm, out_hbm, idx_v, rows_v, sem):
        wid = lax.axis_index("s") * nc + lax.axis_index("c")
        pltpu.sync_copy(idx_hbm.at[pl.ds(wid * nchunk, nchunk)], idx_v)
        copies = [
            pltpu.async_copy(table_hbm.at[idx_v.at[j]],
                             rows_v.at[pl.ds(j * 128, 128)], sem)
            for j in range(nchunk)
        ]
        for c in copies:
            c.wait()
        pltpu.sync_copy(rows_v, out_hbm.at[pl.ds(wid * bw, bw)])

    return gk(table, idx2d)


def kernel(inputs, codebook):
    x = jnp.transpose(inputs, (0, 2, 3, 1))
    flat = x.reshape(-1, _D)
    a2 = jnp.sum(flat * flat, axis=1, keepdims=True)
    b2 = jnp.sum(codebook * codebook, axis=1)[None, :]
    idx2, loss11 = _dist_argmin(flat, codebook, a2, b2)
    quantized = _sc_gather(codebook, idx2.reshape(-1)).reshape(x.shape)
    out = jnp.transpose(quantized, (0, 3, 1, 2))
    mean = loss11[0, 0] / jnp.float32(flat.shape[0] * _D)
    commitment_loss = mean + 0.25 * mean
    return out, commitment_loss
